# baseline (speedup 1.0000x reference)
import functools

import jax
import jax.numpy as jnp
from jax import lax
from jax.experimental import pallas as pl
from jax.experimental.pallas import tpu as pltpu
from jax.experimental.pallas import tpu_sc as plsc

NC = 2
NS = 16
L = 16
NW = NC * NS

B = 16384
D = 64
RELN = 100
BPW = B // NW
CH = 16
NCHK = BPW // CH
RG = 32
QS = 16384.0
INV_S2 = float(2.0 ** -28)

_EPS = 1e-16


def _ln(x):
    bits = lax.bitcast_convert_type(x, jnp.int32)
    e = lax.shift_right_logical(bits, 23) - 126
    m = lax.bitcast_convert_type(
        (bits & jnp.int32(0x007FFFFF)) | jnp.int32(0x3F000000), jnp.float32)
    small = m < jnp.float32(0.7071067811865476)
    m = jnp.where(small, m + m, m) - 1.0
    e = jnp.where(small, e - 1, e)
    ef = e.astype(jnp.float32)
    z = m * m
    p = jnp.float32(7.0376836292e-2)
    for c in (-1.1514610310e-1, 1.1676998740e-1, -1.2420140846e-1,
              1.4249322787e-1, -1.6668057665e-1, 2.0000714765e-1,
              -2.4999993993e-1, 3.3333331174e-1):
        p = p * m + jnp.float32(c)
    y = m * z * p
    y = y + ef * jnp.float32(-2.12194440e-4)
    y = y - 0.5 * z
    return m + y + ef * jnp.float32(0.693359375)


def _sigmoid(x):
    return 1.0 / (1.0 + jnp.exp(-x))


_MESH = plsc.VectorSubcoreMesh(core_axis_name="c", subcore_axis_name="s")


@functools.partial(
    pl.kernel,
    out_type=jax.ShapeDtypeStruct((NW, L), jnp.float32),
    mesh=_MESH,
    scratch_types=[
        pltpu.VMEM((BPW,), jnp.int32),
        pltpu.VMEM((BPW,), jnp.int32),
        pltpu.VMEM((BPW,), jnp.int32),
        pltpu.VMEM((BPW,), jnp.float32),
        pltpu.VMEM((CH, 2, RG, D), jnp.int8),
        pltpu.VMEM((CH, 2, RG, D), jnp.int8),
        pltpu.VMEM((4, D), jnp.float32),
        pltpu.VMEM((RELN, D), jnp.float32),
        pltpu.VMEM((L,), jnp.float32),
        pltpu.VMEM((L,), jnp.float32),
        pltpu.SemaphoreType.DMA,
        pltpu.SemaphoreType.DMA,
    ],
    compiler_params=pltpu.CompilerParams(needs_layout_passes=False),
)
def _sc_loss(attr1, attr2, rel, gt, emb, rel_emb, out,
             idx1_s, idx2_s, rel_s, gt_v,
             blkA_v, blkB_v,
             rstage_v, rtab_v, dots_v, part_v, semA, semB):
    cid = lax.axis_index("c")
    sid = lax.axis_index("s")
    wid = sid * NC + cid
    base = wid * BPW

    pltpu.sync_copy(attr1.at[pl.ds(base, BPW)], idx1_s)
    pltpu.sync_copy(attr2.at[pl.ds(base, BPW)], idx2_s)
    pltpu.sync_copy(rel.at[pl.ds(base, BPW)], rel_s)
    pltpu.sync_copy(gt.at[pl.ds(base, BPW)], gt_v)

    def srow(k, carry):
        pltpu.sync_copy(rel_emb.at[pl.ds(k * 4, 4)], rstage_v)
        for i in range(4):
            for c in range(D // L):
                v = rstage_v[i, pl.ds(c * L, L)]
                rtab_v[k * 4 + i, pl.ds(c * L, L)] = (
                    _sigmoid(v) * INV_S2)
        return carry
    lax.fori_loop(0, RELN // 4, srow, 0)

    lanes = lax.iota(jnp.int32, L)
    last_lane = lanes == (L - 1)

    def fire(ch, blk_v, sem):
        cb = ch * CH
        for g in range(CH // L):
            iv1 = idx1_s[pl.ds(cb + g * L, L)]
            iv2 = idx2_s[pl.ds(cb + g * L, L)]
            for j in range(L):
                e = g * L + j
                b1 = pl.multiple_of((iv1[j] >> 5) << 5, RG)
                b2 = pl.multiple_of((iv2[j] >> 5) << 5, RG)
                pltpu.async_copy(emb.at[pl.ds(b1, RG)], blk_v.at[e, 0], sem)
                pltpu.async_copy(emb.at[pl.ds(b2, RG)], blk_v.at[e, 1], sem)

    def drain_compute(ch, blk_v, sem, total):
        cb = ch * CH
        pltpu.make_async_copy(
            emb.at[pl.ds(0, CH * 2 * RG)], blk_v, sem).wait()
        for g in range(CH // L):
            iv1 = idx1_s[pl.ds(cb + g * L, L)]
            iv2 = idx2_s[pl.ds(cb + g * L, L)]
            wv1 = iv1 & (RG - 1)
            wv2 = iv2 & (RG - 1)
            rv = rel_s[pl.ds(cb + g * L, L)]
            for j in range(L):
                e = g * L + j
                t = jnp.zeros((L,), jnp.float32)
                w1 = plsc.bitcast(blk_v[e, 0, wv1[j], pl.ds(0, D)],
                                  jnp.int32)
                w2 = plsc.bitcast(blk_v[e, 1, wv2[j], pl.ds(0, D)],
                                  jnp.int32)
                for off in range(4):
                    sh = 24 - 8 * off
                    f1 = lax.shift_right_arithmetic(
                        lax.shift_left(w1, sh), 24).astype(jnp.float32)
                    f2 = lax.shift_right_arithmetic(
                        lax.shift_left(w2, sh), 24).astype(jnp.float32)
                    dcol = 4 * lanes + off
                    sr = plsc.load_gather(rtab_v, [rv[j] + 0 * lanes, dcol])
                    t = t + f1 * f2 * sr
                cs = plsc.cumsum(t)
                plsc.store_scatter(dots_v,
                                   [jnp.full((L,), j, jnp.int32)], cs,
                                   mask=last_lane)
            acc = dots_v[...]
            gv = gt_v[pl.ds(cb + g * L, L)]
            pred = _sigmoid(acc)
            loss = -(gv * _ln(pred + _EPS)
                     + (1.0 - gv) * _ln(1.0 - pred + _EPS))
            total = total + loss
        return total

    def pair(k, total):
        fire(2 * k, blkA_v, semA)
        fire(2 * k + 1, blkB_v, semB)
        total = drain_compute(2 * k, blkA_v, semA, total)
        total = drain_compute(2 * k + 1, blkB_v, semB, total)
        return total

    total = lax.fori_loop(0, NCHK // 2, pair, jnp.zeros((L,), jnp.float32))
    part_v[...] = total
    pltpu.sync_copy(part_v, out.at[wid])


TP = 7680


def _cast_body(x_ref, o_ref):
    t = x_ref[...].T
    q = jnp.clip(jnp.round(t * QS), -127.0, 127.0)
    o_ref[...] = q.astype(jnp.int8)


_cast_t = pl.pallas_call(
    _cast_body,
    grid=((1000000 + TP - 1) // TP,),
    in_specs=[pl.BlockSpec((D, TP), lambda j: (0, j))],
    out_specs=pl.BlockSpec((TP, D), lambda j: (j, 0)),
    out_shape=jax.ShapeDtypeStruct((1000000, D), jnp.int8),
)


def _sum_body(x_ref, o_ref):
    o_ref[0, 0] = jnp.sum(x_ref[...])


_reduce = pl.pallas_call(
    _sum_body,
    out_shape=jax.ShapeDtypeStruct((1, 1), jnp.float32),
    out_specs=pl.BlockSpec(memory_space=pltpu.SMEM),
)


def kernel(attr1, attr2, rel, ground_truth, embeddings, relation_embedding):
    part = _sc_loss(attr1, attr2, rel, ground_truth,
                    _cast_t(embeddings.T), relation_embedding)
    return _reduce(part)[0, 0]


# speedup vs baseline: 1.1728x; 1.1728x over previous
import functools

import jax
import jax.numpy as jnp
from jax import lax
from jax.experimental import pallas as pl
from jax.experimental.pallas import tpu as pltpu
from jax.experimental.pallas import tpu_sc as plsc

NC = 2
NS = 16
L = 16
NW = NC * NS

B = 16384
D = 64
RELN = 100
BPW = B // NW
CH = 32
NCHK = BPW // CH
RG = 16

_EPS = 1e-16


def _ln(x):
    bits = lax.bitcast_convert_type(x, jnp.int32)
    e = lax.shift_right_logical(bits, 23) - 126
    m = lax.bitcast_convert_type(
        (bits & jnp.int32(0x007FFFFF)) | jnp.int32(0x3F000000), jnp.float32)
    small = m < jnp.float32(0.7071067811865476)
    m = jnp.where(small, m + m, m) - 1.0
    e = jnp.where(small, e - 1, e)
    ef = e.astype(jnp.float32)
    z = m * m
    p = jnp.float32(7.0376836292e-2)
    for c in (-1.1514610310e-1, 1.1676998740e-1, -1.2420140846e-1,
              1.4249322787e-1, -1.6668057665e-1, 2.0000714765e-1,
              -2.4999993993e-1, 3.3333331174e-1):
        p = p * m + jnp.float32(c)
    y = m * z * p
    y = y + ef * jnp.float32(-2.12194440e-4)
    y = y - 0.5 * z
    return m + y + ef * jnp.float32(0.693359375)


def _sigmoid(x):
    return 1.0 / (1.0 + jnp.exp(-x))


_MESH = plsc.VectorSubcoreMesh(core_axis_name="c", subcore_axis_name="s")


@functools.partial(
    pl.kernel,
    out_type=jax.ShapeDtypeStruct((NW, L), jnp.float32),
    mesh=_MESH,
    scratch_types=[
        pltpu.VMEM((BPW,), jnp.int32),
        pltpu.VMEM((BPW,), jnp.int32),
        pltpu.VMEM((BPW,), jnp.int32),
        pltpu.VMEM((BPW,), jnp.float32),
        pltpu.VMEM((CH, RG, D), jnp.bfloat16),
        pltpu.VMEM((CH, RG, D), jnp.bfloat16),
        pltpu.VMEM((4, D), jnp.float32),
        pltpu.VMEM((RELN, D), jnp.float32),
        pltpu.VMEM((L,), jnp.float32),
        pltpu.VMEM((L,), jnp.float32),
        pltpu.SemaphoreType.DMA,
    ],
    compiler_params=pltpu.CompilerParams(needs_layout_passes=False),
)
def _sc_loss(attr1, attr2, rel, gt, emb, rel_emb, out,
             idx1_s, idx2_s, rel_s, gt_v,
             blk1_v, blk2_v, rstage_v, rtab_v, dots_v, part_v, sem):
    cid = lax.axis_index("c")
    sid = lax.axis_index("s")
    wid = sid * NC + cid
    base = wid * BPW

    pltpu.sync_copy(attr1.at[pl.ds(base, BPW)], idx1_s)
    pltpu.sync_copy(attr2.at[pl.ds(base, BPW)], idx2_s)
    pltpu.sync_copy(rel.at[pl.ds(base, BPW)], rel_s)
    pltpu.sync_copy(gt.at[pl.ds(base, BPW)], gt_v)

    def srow(k, carry):
        pltpu.sync_copy(rel_emb.at[pl.ds(k * 4, 4)], rstage_v)
        for i in range(4):
            for c in range(D // L):
                v = rstage_v[i, pl.ds(c * L, L)]
                rtab_v[k * 4 + i, pl.ds(c * L, L)] = _sigmoid(v)
        return carry
    lax.fori_loop(0, RELN // 4, srow, 0)

    lanes = lax.iota(jnp.int32, L)
    last_lane = lanes == (L - 1)

    def chunk(ch, total):
        cb = ch * CH
        ivs = []
        for g in range(CH // L):
            iv1 = idx1_s[pl.ds(cb + g * L, L)]
            iv2 = idx2_s[pl.ds(cb + g * L, L)]
            ivs.append((iv1, iv2))
            for j in range(L):
                e = g * L + j
                b1 = pl.multiple_of((iv1[j] >> 4) << 4, RG)
                b2 = pl.multiple_of((iv2[j] >> 4) << 4, RG)
                pltpu.async_copy(emb.at[pl.ds(b1, RG)], blk1_v.at[e], sem)
                pltpu.async_copy(emb.at[pl.ds(b2, RG)], blk2_v.at[e], sem)

        pltpu.make_async_copy(
            emb.at[pl.ds(0, CH * RG)], blk1_v, sem).wait()
        pltpu.make_async_copy(
            emb.at[pl.ds(0, CH * RG)], blk2_v, sem).wait()

        for g in range(CH // L):
            iv1, iv2 = ivs[g]
            wv1 = iv1 & (RG - 1)
            wv2 = iv2 & (RG - 1)
            rv = rel_s[pl.ds(cb + g * L, L)]
            for j in range(L):
                e = g * L + j
                t = jnp.zeros((L,), jnp.float32)
                for h in range(2):
                    q1 = blk1_v[e, wv1[j], pl.ds(h * 32, 32)]
                    q2 = blk2_v[e, wv2[j], pl.ds(h * 32, 32)]
                    e1a, e1b = plsc.unpack(q1,
                                           format=plsc.PackFormat.INTERLEAVED)
                    e2a, e2b = plsc.unpack(q2,
                                           format=plsc.PackFormat.INTERLEAVED)
                    da = jnp.full((L,), h * 32, jnp.int32) + 2 * lanes
                    sra = plsc.load_gather(rtab_v, [rv[j] + 0 * lanes, da])
                    srb = plsc.load_gather(rtab_v, [rv[j] + 0 * lanes,
                                                    da + 1])
                    t = t + e1a * e2a * sra + e1b * e2b * srb
                cs = plsc.cumsum(t)
                plsc.store_scatter(dots_v,
                                   [jnp.full((L,), j, jnp.int32)], cs,
                                   mask=last_lane)
            acc = dots_v[...]
            gv = gt_v[pl.ds(cb + g * L, L)]
            pred = _sigmoid(acc)
            loss = -(gv * _ln(pred + _EPS)
                     + (1.0 - gv) * _ln(1.0 - pred + _EPS))
            total = total + loss
        return total

    total = lax.fori_loop(0, NCHK, chunk, jnp.zeros((L,), jnp.float32))
    part_v[...] = total
    pltpu.sync_copy(part_v, out.at[wid])


TP = 15488


def _cast_body(x_ref, o_ref):
    o_ref[...] = x_ref[...].astype(jnp.bfloat16).T


_cast_t = pl.pallas_call(
    _cast_body,
    grid=((1000000 + TP - 1) // TP,),
    in_specs=[pl.BlockSpec((D, TP), lambda j: (0, j))],
    out_specs=pl.BlockSpec((TP, D), lambda j: (j, 0)),
    out_shape=jax.ShapeDtypeStruct((1000000, D), jnp.bfloat16),
)


def _sum_body(x_ref, o_ref):
    o_ref[0, 0] = jnp.sum(x_ref[...])


_reduce = pl.pallas_call(
    _sum_body,
    out_shape=jax.ShapeDtypeStruct((1, 1), jnp.float32),
    out_specs=pl.BlockSpec(memory_space=pltpu.SMEM),
)


def kernel(attr1, attr2, rel, ground_truth, embeddings, relation_embedding):
    part = _sc_loss(attr1, attr2, rel, ground_truth,
                    _cast_t(embeddings.T), relation_embedding)
    return _reduce(part)[0, 0]


# speedup vs baseline: 1.1975x; 1.0211x over previous
import functools

import jax
import jax.numpy as jnp
from jax import lax
from jax.experimental import pallas as pl
from jax.experimental.pallas import tpu as pltpu
from jax.experimental.pallas import tpu_sc as plsc

NC = 2
NS = 16
L = 16
NW = NC * NS

B = 16384
D = 64
RELN = 100
BPW = B // NW
CH = 32
NCHK = BPW // CH
RG = 16

_EPS = 1e-16


def _ln(x):
    bits = lax.bitcast_convert_type(x, jnp.int32)
    e = lax.shift_right_logical(bits, 23) - 126
    m = lax.bitcast_convert_type(
        (bits & jnp.int32(0x007FFFFF)) | jnp.int32(0x3F000000), jnp.float32)
    small = m < jnp.float32(0.7071067811865476)
    m = jnp.where(small, m + m, m) - 1.0
    e = jnp.where(small, e - 1, e)
    ef = e.astype(jnp.float32)
    z = m * m
    p = jnp.float32(7.0376836292e-2)
    for c in (-1.1514610310e-1, 1.1676998740e-1, -1.2420140846e-1,
              1.4249322787e-1, -1.6668057665e-1, 2.0000714765e-1,
              -2.4999993993e-1, 3.3333331174e-1):
        p = p * m + jnp.float32(c)
    y = m * z * p
    y = y + ef * jnp.float32(-2.12194440e-4)
    y = y - 0.5 * z
    return m + y + ef * jnp.float32(0.693359375)


def _sigmoid(x):
    return 1.0 / (1.0 + jnp.exp(-x))


_MESH = plsc.VectorSubcoreMesh(core_axis_name="c", subcore_axis_name="s")


@functools.partial(
    pl.kernel,
    out_type=jax.ShapeDtypeStruct((NW, L), jnp.float32),
    mesh=_MESH,
    scratch_types=[
        pltpu.VMEM((BPW,), jnp.int32),
        pltpu.VMEM((BPW,), jnp.int32),
        pltpu.VMEM((BPW,), jnp.int32),
        pltpu.VMEM((BPW,), jnp.float32),
        pltpu.VMEM((CH, RG, D), jnp.bfloat16),
        pltpu.VMEM((CH, RG, D), jnp.bfloat16),
        pltpu.VMEM((4, D), jnp.float32),
        pltpu.VMEM((RELN, D), jnp.float32),
        pltpu.VMEM((L,), jnp.float32),
        pltpu.VMEM((L,), jnp.float32),
        pltpu.SemaphoreType.DMA,
    ],
    compiler_params=pltpu.CompilerParams(needs_layout_passes=False),
)
def _sc_loss(attr1, attr2, rel, gt, emb, rel_emb, out,
             idx1_s, idx2_s, rel_s, gt_v,
             blk1_v, blk2_v, rstage_v, rtab_v, dots_v, part_v, sem):
    cid = lax.axis_index("c")
    sid = lax.axis_index("s")
    wid = sid * NC + cid
    base = wid * BPW

    pltpu.sync_copy(attr1.at[pl.ds(base, BPW)], idx1_s)
    pltpu.sync_copy(attr2.at[pl.ds(base, BPW)], idx2_s)
    pltpu.sync_copy(rel.at[pl.ds(base, BPW)], rel_s)
    pltpu.sync_copy(gt.at[pl.ds(base, BPW)], gt_v)

    def srow(k, carry):
        pltpu.sync_copy(rel_emb.at[pl.ds(k * 4, 4)], rstage_v)
        for i in range(4):
            for c in range(D // L):
                v = rstage_v[i, pl.ds(c * L, L)]
                rtab_v[k * 4 + i, pl.ds(c * L, L)] = _sigmoid(v)
        return carry
    lax.fori_loop(0, RELN // 4, srow, 0)

    lanes = lax.iota(jnp.int32, L)
    last_lane = lanes == (L - 1)

    def chunk(ch, total):
        cb = ch * CH
        ivs = []
        for g in range(CH // L):
            iv1 = idx1_s[pl.ds(cb + g * L, L)]
            iv2 = idx2_s[pl.ds(cb + g * L, L)]
            ivs.append((iv1, iv2))
            for j in range(L):
                e = g * L + j
                b1 = pl.multiple_of((iv1[j] >> 4) << 4, RG)
                b2 = pl.multiple_of((iv2[j] >> 4) << 4, RG)
                pltpu.async_copy(emb.at[pl.ds(b1, RG)], blk1_v.at[e], sem)
                pltpu.async_copy(emb.at[pl.ds(b2, RG)], blk2_v.at[e], sem)

        pltpu.make_async_copy(
            emb.at[pl.ds(0, CH * RG)], blk1_v, sem).wait()
        pltpu.make_async_copy(
            emb.at[pl.ds(0, CH * RG)], blk2_v, sem).wait()

        for g in range(CH // L):
            iv1, iv2 = ivs[g]
            wv1 = iv1 & (RG - 1)
            wv2 = iv2 & (RG - 1)
            rv = rel_s[pl.ds(cb + g * L, L)]
            for j in range(L):
                e = g * L + j
                t = jnp.zeros((L,), jnp.float32)
                for h in range(2):
                    q1 = blk1_v[e, wv1[j], pl.ds(h * 32, 32)]
                    q2 = blk2_v[e, wv2[j], pl.ds(h * 32, 32)]
                    e1a, e1b = plsc.unpack(q1,
                                           format=plsc.PackFormat.INTERLEAVED)
                    e2a, e2b = plsc.unpack(q2,
                                           format=plsc.PackFormat.INTERLEAVED)
                    da = jnp.full((L,), h * 32, jnp.int32) + 2 * lanes
                    sra = plsc.load_gather(rtab_v, [rv[j] + 0 * lanes, da])
                    srb = plsc.load_gather(rtab_v, [rv[j] + 0 * lanes,
                                                    da + 1])
                    t = t + e1a * e2a * sra + e1b * e2b * srb
                cs = plsc.cumsum(t)
                plsc.store_scatter(dots_v,
                                   [jnp.full((L,), j, jnp.int32)], cs,
                                   mask=last_lane)
            acc = dots_v[...]
            gv = gt_v[pl.ds(cb + g * L, L)]
            pred = _sigmoid(acc)
            loss = -(gv * _ln(pred + _EPS)
                     + (1.0 - gv) * _ln(1.0 - pred + _EPS))
            total = total + loss
        return total

    total = lax.fori_loop(0, NCHK, chunk, jnp.zeros((L,), jnp.float32))
    part_v[...] = total
    pltpu.sync_copy(part_v, out.at[wid])


TP = 31104


def _cast_body(x_ref, o_ref):
    o_ref[...] = x_ref[...].astype(jnp.bfloat16).T


_cast_t = pl.pallas_call(
    _cast_body,
    grid=((1000000 + TP - 1) // TP,),
    in_specs=[pl.BlockSpec((D, TP), lambda j: (0, j))],
    out_specs=pl.BlockSpec((TP, D), lambda j: (j, 0)),
    out_shape=jax.ShapeDtypeStruct((1000000, D), jnp.bfloat16),
)


def _sum_body(x_ref, o_ref):
    o_ref[0, 0] = jnp.sum(x_ref[...])


_reduce = pl.pallas_call(
    _sum_body,
    out_shape=jax.ShapeDtypeStruct((1, 1), jnp.float32),
    out_specs=pl.BlockSpec(memory_space=pltpu.SMEM),
)


def kernel(attr1, attr2, rel, ground_truth, embeddings, relation_embedding):
    part = _sc_loss(attr1, attr2, rel, ground_truth,
                    _cast_t(embeddings.T), relation_embedding)
    return _reduce(part)[0, 0]
